# baseline (speedup 1.0000x reference)
import functools

import jax
import jax.numpy as jnp
from jax import lax
from jax.experimental import pallas as pl
from jax.experimental.pallas import tpu as pltpu
from jax.experimental.pallas import tpu_sc as plsc

NN = 10000
EE = 320000
HID = 128
NBLK = 2000
EBLK = 2000

_HI = jax.lax.Precision.HIGHEST


def _dot(a, b):
    return jnp.dot(a, b, precision=_HI, preferred_element_type=jnp.float32)


def _sigmoid(x):
    return 1.0 / (1.0 + jnp.exp(-x))


def _rtne16(x):
    xi = pltpu.bitcast(x, jnp.int32)
    r = xi + jnp.int32(0x7FFF) + jnp.bitwise_and(
        jax.lax.shift_right_logical(xi, 16), jnp.int32(1))
    return jax.lax.shift_right_logical(r, 16)


def _pack2(a, b):
    return jnp.bitwise_or(_rtne16(a),
                          jax.lax.shift_left(_rtne16(b), jnp.int32(16)))


def _unpack2(p):
    hi = jax.lax.shift_left(jax.lax.shift_right_logical(p, 16),
                            jnp.int32(16))
    lo = jax.lax.shift_left(p, jnp.int32(16))
    return (pltpu.bitcast(lo, jnp.float32),
            pltpu.bitcast(hi, jnp.float32))


def _prep_body(h_ref, WhW, Whb, AW, Ab, BW, Bb, DW, Db, EW, Eb, eW, eb,
               C1W, C1b, h1_o, Ah_o, Tsrc_o, Tdst_o, W1_o, c1_o):
    h1 = _dot(h_ref[...], WhW[...]) + Whb[...]
    h1_o[...] = h1
    Ah_o[...] = _dot(h1, AW[...]) + Ab[...]
    dh = _dot(h1, DW[...]) + Db[...]
    bh = _dot(h1, BW[...]) + Bb[...]
    Tsrc_o[...] = _pack2(dh, bh)
    eh = _dot(h1, EW[...]) + Eb[...]
    ehp = eh.reshape(eh.shape[0] // 2, 2, HID)
    Tdst_o[...] = _pack2(ehp[:, 0, :], ehp[:, 1, :])
    W1_o[...] = _dot(eW[...], C1W[...])
    c1_o[...] = jnp.broadcast_to(_dot(eb[...], C1W[...]) + C1b[...], (8, HID))


def _prep(h, p, lp):
    nb = NN // NBLK
    full = lambda s: pl.BlockSpec(s, lambda i: (0, 0))
    blk = lambda s: pl.BlockSpec(s, lambda i: (i, 0))
    w128 = full((HID, HID))
    b128 = full((1, HID))
    ins = [blk((NBLK, HID))] + [w128, b128] * 5 + [full((16, HID)), full((1, HID)), w128, b128]
    outs = [blk((NBLK, HID)), blk((NBLK, HID)), blk((NBLK, HID)),
            blk((NBLK // 2, HID)), full((16, HID)), full((8, HID))]
    oshapes = [jax.ShapeDtypeStruct((NN, HID), jnp.float32),
               jax.ShapeDtypeStruct((NN, HID), jnp.float32),
               jax.ShapeDtypeStruct((NN, HID), jnp.int32),
               jax.ShapeDtypeStruct((NN // 2, HID), jnp.int32),
               jax.ShapeDtypeStruct((16, HID), jnp.float32),
               jax.ShapeDtypeStruct((8, HID), jnp.float32)]
    r = lambda v: v.reshape(1, -1)
    return pl.pallas_call(
        _prep_body, grid=(nb,), in_specs=ins, out_specs=outs, out_shape=oshapes,
    )(h, p['emb_h_W'], r(p['emb_h_b']),
      lp['A_W'], r(lp['A_b']), lp['B_W'], r(lp['B_b']),
      lp['D_W'], r(lp['D_b']), lp['E_W'], r(lp['E_b']),
      p['emb_e_W'], r(p['emb_e_b']), lp['C_W'], r(lp['C_b']))


def _edge1_body(e_ref, gs_ref, gd_ref, dp_ref, W1, c1, enew_o, sbsig_o,
                stats_o, acc):
    i = pl.program_id(0)

    @pl.when(i == 0)
    def _():
        acc[...] = jnp.zeros_like(acc)

    ce = _dot(e_ref[...], W1[...]) + c1[0:1, :]
    dh, bh = _unpack2(gs_ref[...])
    ev, od = _unpack2(gd_ref[...])
    eh = jnp.where(dp_ref[...] == 0, ev, od)
    e_new = ce + dh + eh
    enew_o[...] = e_new
    sg = _sigmoid(e_new)
    sbsig_o[:, HID:] = sg
    sbsig_o[:, :HID] = sg * bh
    acc[0:1, :] += jnp.sum(e_new, axis=0, keepdims=True)
    acc[1:2, :] += jnp.sum(e_new * e_new, axis=0, keepdims=True)

    @pl.when(i == pl.num_programs(0) - 1)
    def _():
        stats_o[...] = acc[...]


def _edge1(e, gsrc, gdst, dpar, W1, c1):
    ne = e.shape[0]
    nb = ne // EBLK
    blk = lambda s: pl.BlockSpec(s, lambda i: (i, 0))
    full = lambda s: pl.BlockSpec(s, lambda i: (0, 0))
    return pl.pallas_call(
        _edge1_body, grid=(nb,),
        in_specs=[blk((EBLK, 16)), blk((EBLK, HID)), blk((EBLK, HID)),
                  blk((EBLK, 1)), full((16, HID)), full((8, HID))],
        out_specs=[blk((EBLK, HID)), blk((EBLK, 2 * HID)), full((8, HID))],
        out_shape=[jax.ShapeDtypeStruct((ne, HID), jnp.float32),
                   jax.ShapeDtypeStruct((ne, 2 * HID), jnp.float32),
                   jax.ShapeDtypeStruct((8, HID), jnp.float32)],
        scratch_shapes=[pltpu.VMEM((8, HID), jnp.float32)],
    )(e, gsrc, gdst, dpar, W1, c1)


def _nodea_body(Ah_ref, numa_ref, dena_ref, numb_ref, denb_ref,
                hpre_o, stats_o, acc):
    i = pl.program_id(0)

    @pl.when(i == 0)
    def _():
        acc[...] = jnp.zeros_like(acc)

    num = numa_ref[...] + numb_ref[...]
    den = dena_ref[...] + denb_ref[...]
    hp = Ah_ref[...] + num / (den + 1e-6)
    hpre_o[...] = hp
    acc[0:1, :] += jnp.sum(hp, axis=0, keepdims=True)
    acc[1:2, :] += jnp.sum(hp * hp, axis=0, keepdims=True)

    @pl.when(i == pl.num_programs(0) - 1)
    def _():
        stats_o[...] = acc[...]


def _nodea(Ah, numa, dena, numb, denb):
    nb = NN // NBLK
    blk = lambda s: pl.BlockSpec(s, lambda i: (i, 0))
    full = lambda s: pl.BlockSpec(s, lambda i: (0, 0))
    return pl.pallas_call(
        _nodea_body, grid=(nb,),
        in_specs=[blk((NBLK, HID))] * 5,
        out_specs=[blk((NBLK, HID)), full((8, HID))],
        out_shape=[jax.ShapeDtypeStruct((NN, HID), jnp.float32),
                   jax.ShapeDtypeStruct((8, HID), jnp.float32)],
        scratch_shapes=[pltpu.VMEM((8, HID), jnp.float32)],
    )(Ah, numa, dena, numb, denb)


def _nodeb_body(hpre_ref, h1_ref, stats_ref, g_ref, b_ref,
                AW, Ab, BW, Bb, DW, Db, EW, Eb,
                h2_o, Ah_o, Tsrc_o, Tdst_o):
    s = stats_ref[0:1, :]
    q = stats_ref[1:2, :]
    mu = s / NN
    var = q / NN - mu * mu
    rstd = jax.lax.rsqrt(var + 1e-5)
    hn = (hpre_ref[...] - mu) * rstd * g_ref[...] + b_ref[...]
    h2 = jnp.maximum(hn, 0.0) + h1_ref[...]
    h2_o[...] = h2
    Ah_o[...] = _dot(h2, AW[...]) + Ab[...]
    dh = _dot(h2, DW[...]) + Db[...]
    bh = _dot(h2, BW[...]) + Bb[...]
    Tsrc_o[...] = _pack2(dh, bh)
    eh = _dot(h2, EW[...]) + Eb[...]
    ehp = eh.reshape(eh.shape[0] // 2, 2, HID)
    Tdst_o[...] = _pack2(ehp[:, 0, :], ehp[:, 1, :])


def _nodeb_mats(hpre, h1, stats, g, b, lp):
    nb = NN // NBLK
    blk = lambda s: pl.BlockSpec(s, lambda i: (i, 0))
    full = lambda s: pl.BlockSpec(s, lambda i: (0, 0))
    w128 = full((HID, HID))
    b128 = full((1, HID))
    r = lambda v: v.reshape(1, -1)
    return pl.pallas_call(
        _nodeb_body, grid=(nb,),
        in_specs=[blk((NBLK, HID)), blk((NBLK, HID)), full((8, HID)),
                  b128, b128] + [w128, b128] * 4,
        out_specs=[blk((NBLK, HID)), blk((NBLK, HID)), blk((NBLK, HID)),
                   blk((NBLK // 2, HID))],
        out_shape=[jax.ShapeDtypeStruct((NN, HID), jnp.float32),
                   jax.ShapeDtypeStruct((NN, HID), jnp.float32),
                   jax.ShapeDtypeStruct((NN, HID), jnp.int32),
                   jax.ShapeDtypeStruct((NN // 2, HID), jnp.int32)],
    )(hpre, h1, stats, r(g), r(b),
      lp['A_W'], r(lp['A_b']), lp['B_W'], r(lp['B_b']),
      lp['D_W'], r(lp['D_b']), lp['E_W'], r(lp['E_b']))


def _nodeb_final_body(hpre_ref, h1_ref, stats_ref, g_ref, b_ref, h2_o):
    s = stats_ref[0:1, :]
    q = stats_ref[1:2, :]
    mu = s / NN
    var = q / NN - mu * mu
    rstd = jax.lax.rsqrt(var + 1e-5)
    hn = (hpre_ref[...] - mu) * rstd * g_ref[...] + b_ref[...]
    h2_o[...] = jnp.maximum(hn, 0.0) + h1_ref[...]


def _nodeb_final(hpre, h1, stats, g, b):
    nb = NN // NBLK
    blk = lambda s: pl.BlockSpec(s, lambda i: (i, 0))
    full = lambda s: pl.BlockSpec(s, lambda i: (0, 0))
    r = lambda v: v.reshape(1, -1)
    return pl.pallas_call(
        _nodeb_final_body, grid=(nb,),
        in_specs=[blk((NBLK, HID)), blk((NBLK, HID)), full((8, HID)),
                  full((1, HID)), full((1, HID))],
        out_specs=blk((NBLK, HID)),
        out_shape=jax.ShapeDtypeStruct((NN, HID), jnp.float32),
    )(hpre, h1, stats, r(g), r(b))


def _edge2_body(enew1_ref, e_ref, gs_ref, gd_ref, dp_ref, statsa_ref,
                statsb_ref, g1_ref, b1_ref, eW, ebias, C2W, C2b, sbsig_o):
    s = statsa_ref[0:1, :] + statsb_ref[0:1, :]
    q = statsa_ref[1:2, :] + statsb_ref[1:2, :]
    mu = s / EE
    var = q / EE - mu * mu
    rstd = jax.lax.rsqrt(var + 1e-5)
    gnorm = (enew1_ref[...] - mu) * rstd * g1_ref[...] + b1_ref[...]
    g1 = jnp.maximum(gnorm, 0.0)
    e1 = _dot(e_ref[...], eW[...]) + ebias[...]
    e_out1 = g1 + e1
    ce2 = _dot(e_out1, C2W[...]) + C2b[...]
    dh, bh = _unpack2(gs_ref[...])
    ev, od = _unpack2(gd_ref[...])
    eh = jnp.where(dp_ref[...] == 0, ev, od)
    e_new2 = ce2 + dh + eh
    sg = _sigmoid(e_new2)
    sbsig_o[:, HID:] = sg
    sbsig_o[:, :HID] = sg * bh


def _edge2(enew1, e, gsrc, gdst, dpar, statsa, statsb, p, lp1, lp2):
    ne = e.shape[0]
    nb = ne // EBLK
    blk = lambda s: pl.BlockSpec(s, lambda i: (i, 0))
    full = lambda s: pl.BlockSpec(s, lambda i: (0, 0))
    r = lambda v: v.reshape(1, -1)
    return pl.pallas_call(
        _edge2_body, grid=(nb,),
        in_specs=[blk((EBLK, HID)), blk((EBLK, 16)), blk((EBLK, HID)),
                  blk((EBLK, HID)), blk((EBLK, 1)), full((8, HID)),
                  full((8, HID)), full((1, HID)), full((1, HID)),
                  full((16, HID)), full((1, HID)), full((HID, HID)),
                  full((1, HID))],
        out_specs=blk((EBLK, 2 * HID)),
        out_shape=jax.ShapeDtypeStruct((ne, 2 * HID), jnp.float32),
    )(enew1, e, gsrc, gdst, dpar, statsa, statsb, r(lp1['bn_e_g']),
      r(lp1['bn_e_b']), p['emb_e_W'], r(p['emb_e_b']),
      lp2['C_W'], r(lp2['C_b']))


_NW = 32
_EPW = EE // _NW
_CHG = 40
_GROWS = _EPW // _CHG
_CHS = 80
_SROWS = (EE // 16) // _CHS


def _sc_mesh():
    return plsc.VectorSubcoreMesh(core_axis_name="c", subcore_axis_name="s")


def _pipeline2(nslots, issue_in, wait_in, issue_out, wait_out):
    def slot(i, b):
        @pl.when(i >= 2)
        def _():
            wait_out(b)

        issue_in(i, b)

        @pl.when(i >= 1)
        def _():
            wait_in(1 - b)
            issue_out(i - 1, 1 - b)

    issue_in(0, 0)

    def body(k, _):
        slot(1 + 2 * k, 1)
        slot(2 + 2 * k, 0)
        return 0

    lax.fori_loop(0, (nslots - 1) // 2, body, 0)
    last = nslots - 1
    if nslots % 2 == 0:
        wait_out(1)
        issue_in(last, 1)
        wait_in(0)
        issue_out(last - 1, 0)
        lastb = 1
    else:
        lastb = 0
    wait_in(lastb)
    issue_out(last, lastb)
    wait_out(1 - lastb)
    wait_out(lastb)


def _gather_sc_body(epw, Tsrc_hbm, Tdst_hbm, src_hbm, dst_hbm,
                    gs_out, gd_out,
                    idxs, idxd, bs0, bs1, bd0, bd1,
                    sgs0, sgs1, sgd0, sgd1, sws0, sws1, swd0, swd1,
                    Ts_sp):
    c = lax.axis_index("c")
    s = lax.axis_index("s")
    wid = s * 2 + c
    base = wid * epw

    @pl.when(s == 0)
    def _():
        pltpu.sync_copy(Tsrc_hbm, Ts_sp)

    pltpu.sync_copy(src_hbm.at[pl.ds(base, epw)], idxs)
    pltpu.sync_copy(dst_hbm.at[pl.ds(base, epw)], idxd)
    plsc.subcore_barrier()
    bs = (bs0, bs1)
    bd = (bd0, bd1)
    sgs = (sgs0, sgs1)
    sgd = (sgd0, sgd1)
    sws = (sws0, sws1)
    swd = (swd0, swd1)

    def issue_in(i, b):
        sl = pl.ds(i * _CHG, _CHG)
        pltpu.async_copy(Ts_sp.at[idxs.at[sl]], bs[b], sgs[b])
        pltpu.async_copy(Tdst_hbm.at[idxd.at[sl]], bd[b], sgd[b])

    def wait_in(b):
        sl = pl.ds(0, _CHG)
        pltpu.make_async_copy(Ts_sp.at[idxs.at[sl]], bs[b], sgs[b]).wait()
        pltpu.make_async_copy(Tdst_hbm.at[idxd.at[sl]], bd[b], sgd[b]).wait()

    def issue_out(i, b):
        off = base + i * _CHG
        pltpu.async_copy(bs[b], gs_out.at[pl.ds(off, _CHG)], sws[b])
        pltpu.async_copy(bd[b], gd_out.at[pl.ds(off, _CHG)], swd[b])

    def wait_out(b):
        pltpu.make_async_copy(bs[b], gs_out.at[pl.ds(0, _CHG)], sws[b]).wait()
        pltpu.make_async_copy(bd[b], gd_out.at[pl.ds(0, _CHG)], swd[b]).wait()

    _pipeline2(epw // _CHG, issue_in, wait_in, issue_out, wait_out)


def _gather(Tsrc, Tdst, src, dst):
    ne = src.shape[0]
    epw = ne // _NW
    f = functools.partial(
        pl.kernel,
        out_type=(jax.ShapeDtypeStruct((ne, HID), jnp.int32),
                  jax.ShapeDtypeStruct((ne, HID), jnp.int32)),
        mesh=_sc_mesh(),
        scratch_types=[pltpu.VMEM((epw,), jnp.int32),
                       pltpu.VMEM((epw,), jnp.int32),
                       pltpu.VMEM((_CHG, HID), jnp.int32),
                       pltpu.VMEM((_CHG, HID), jnp.int32),
                       pltpu.VMEM((_CHG, HID), jnp.int32),
                       pltpu.VMEM((_CHG, HID), jnp.int32)]
                      + [pltpu.SemaphoreType.DMA] * 8
                      + [pltpu.VMEM_SHARED((NN, HID), jnp.int32)],
    )(functools.partial(_gather_sc_body, epw))
    return f(Tsrc, Tdst, src, dst)


_NPAD = 10240


def _scatter_sc_body(eps, valsA_hbm, dstA_hbm,
                     zeros_hbm, out_hbm,
                     ix0, ix1, bv0, bv1,
                     sx0, sx1, sl0, sl1, ss0, ss1, acc):
    c = lax.axis_index("c")
    s = lax.axis_index("s")
    stripe = _NPAD // 16
    pltpu.sync_copy(zeros_hbm.at[pl.ds(s * stripe, stripe)],
                    acc.at[pl.ds(s * stripe, stripe)])
    plsc.subcore_barrier()
    ix = (ix0, ix1)
    bv = (bv0, bv1)
    sx = (sx0, sx1)
    sl = (sl0, sl1)
    ss = (ss0, ss1)
    base = s * eps

    def make_stage(vals_hbm, dst_hbm):
        def issue_in(i, b):
            off = base + i * _CHS
            pltpu.async_copy(dst_hbm.at[pl.ds(off, _CHS)], ix[b], sx[b])
            pltpu.async_copy(
                vals_hbm.at[pl.ds(off, _CHS), pl.ds(c * HID, HID)],
                bv[b], sl[b])

        def wait_in(b):
            pltpu.make_async_copy(
                dst_hbm.at[pl.ds(0, _CHS)], ix[b], sx[b]).wait()
            pltpu.make_async_copy(
                vals_hbm.at[pl.ds(0, _CHS), pl.ds(0, HID)],
                bv[b], sl[b]).wait()

        def issue_out(i, b):
            pltpu.async_copy(bv[b], acc.at[ix[b]], ss[b], add=True)

        def wait_out(b):
            pltpu.make_async_copy(bv[b], acc.at[ix[b]], ss[b]).wait()

        return issue_in, wait_in, issue_out, wait_out

    _pipeline2(eps // _CHS, *make_stage(valsA_hbm, dstA_hbm))
    plsc.subcore_barrier()
    pltpu.sync_copy(acc.at[pl.ds(s * stripe, stripe)],
                    out_hbm.at[c, pl.ds(s * stripe, stripe)])


def _scatter(sbsigA, dstA, zeros):
    eps = dstA.shape[0] // 16
    f = functools.partial(
        pl.kernel,
        out_type=jax.ShapeDtypeStruct((2, _NPAD, HID), jnp.float32),
        mesh=_sc_mesh(),
        scratch_types=[pltpu.VMEM((_CHS,), jnp.int32),
                       pltpu.VMEM((_CHS,), jnp.int32),
                       pltpu.VMEM((_CHS, HID), jnp.float32),
                       pltpu.VMEM((_CHS, HID), jnp.float32)]
                      + [pltpu.SemaphoreType.DMA] * 6
                      + [pltpu.VMEM_SHARED((_NPAD, HID), jnp.float32)],
    )(functools.partial(_scatter_sc_body, eps))
    out = f(sbsigA, dstA, zeros)
    return out[0, :NN], out[1, :NN]


def kernel(h, e, edge_index, params):
    src = edge_index[0]
    dst = edge_index[1]
    lp1, lp2 = params['layers']
    zeros = jnp.zeros((_NPAD, HID), jnp.float32)
    HF = EE // 2
    sA, sB = src[:HF], src[HF:]
    dA, dB = dst[:HF], dst[HF:]
    dhA, dhB = dA >> 1, dB >> 1
    dpA = (dA & 1).reshape(HF, 1)
    dpB = (dB & 1).reshape(HF, 1)
    eA, eB = e[:HF], e[HF:]

    h1, Ah1, Tsrc1, Tdst1, W1, c1 = _prep(h, params, lp1)
    g1a = _gather(Tsrc1, Tdst1, sA, dhA)
    g1b = _gather(Tsrc1, Tdst1, sB, dhB)
    en1a, sb1a, st1a = _edge1(eA, g1a[0], g1a[1], dpA, W1, c1)
    en1b, sb1b, st1b = _edge1(eB, g1b[0], g1b[1], dpB, W1, c1)
    n1a, d1a = _scatter(sb1a, dA, zeros)
    n1b, d1b = _scatter(sb1b, dB, zeros)
    hpre1, hstats1 = _nodea(Ah1, n1a, d1a, n1b, d1b)
    h2, Ah2, Tsrc2, Tdst2 = _nodeb_mats(hpre1, h1, hstats1,
                                        lp1['bn_h_g'], lp1['bn_h_b'], lp2)
    g2a = _gather(Tsrc2, Tdst2, sA, dhA)
    g2b = _gather(Tsrc2, Tdst2, sB, dhB)
    sb2a = _edge2(en1a, eA, g2a[0], g2a[1], dpA, st1a, st1b,
                  params, lp1, lp2)
    sb2b = _edge2(en1b, eB, g2b[0], g2b[1], dpB, st1a, st1b,
                  params, lp1, lp2)
    n2a, d2a = _scatter(sb2a, dA, zeros)
    n2b, d2b = _scatter(sb2b, dB, zeros)
    hpre2, hstats2 = _nodea(Ah2, n2a, d2a, n2b, d2b)
    return _nodeb_final(hpre2, h2, hstats2, lp2['bn_h_g'], lp2['bn_h_b'])


# speedup vs baseline: 1.0813x; 1.0813x over previous
import functools

import jax
import jax.numpy as jnp
from jax import lax
from jax.experimental import pallas as pl
from jax.experimental.pallas import tpu as pltpu
from jax.experimental.pallas import tpu_sc as plsc

NN = 10000
EE = 320000
HID = 128
NBLK = 2000
EBLK = 2000

_HI = jax.lax.Precision.HIGHEST


def _dot(a, b):
    return jnp.dot(a, b, precision=_HI, preferred_element_type=jnp.float32)


def _sigmoid(x):
    return 1.0 / (1.0 + jnp.exp(-x))


def _rtne16(x):
    xi = pltpu.bitcast(x, jnp.int32)
    r = xi + jnp.int32(0x7FFF) + jnp.bitwise_and(
        jax.lax.shift_right_logical(xi, 16), jnp.int32(1))
    return jax.lax.shift_right_logical(r, 16)


def _pack2(a, b):
    return jnp.bitwise_or(_rtne16(a),
                          jax.lax.shift_left(_rtne16(b), jnp.int32(16)))


def _unpack2(p):
    hi = jax.lax.shift_left(jax.lax.shift_right_logical(p, 16),
                            jnp.int32(16))
    lo = jax.lax.shift_left(p, jnp.int32(16))
    return (pltpu.bitcast(lo, jnp.float32),
            pltpu.bitcast(hi, jnp.float32))


def _prep_body(h_ref, WhW, Whb, AW, Ab, BW, Bb, DW, Db, EW, Eb, eW, eb,
               C1W, C1b, h1_o, Ah_o, Tsrc_o, Tdst_o, W1_o, c1_o):
    h1 = _dot(h_ref[...], WhW[...]) + Whb[...]
    h1_o[...] = h1
    Ah_o[...] = _dot(h1, AW[...]) + Ab[...]
    dh = _dot(h1, DW[...]) + Db[...]
    bh = _dot(h1, BW[...]) + Bb[...]
    Tsrc_o[...] = _pack2(dh, bh)
    Tdst_o[...] = _dot(h1, EW[...]) + Eb[...]
    W1_o[...] = _dot(eW[...], C1W[...])
    c1_o[...] = jnp.broadcast_to(_dot(eb[...], C1W[...]) + C1b[...], (8, HID))


def _prep(h, p, lp):
    nb = NN // NBLK
    full = lambda s: pl.BlockSpec(s, lambda i: (0, 0))
    blk = lambda s: pl.BlockSpec(s, lambda i: (i, 0))
    w128 = full((HID, HID))
    b128 = full((1, HID))
    ins = [blk((NBLK, HID))] + [w128, b128] * 5 + [full((16, HID)), full((1, HID)), w128, b128]
    outs = [blk((NBLK, HID)), blk((NBLK, HID)), blk((NBLK, HID)),
            blk((NBLK, HID)), full((16, HID)), full((8, HID))]
    oshapes = [jax.ShapeDtypeStruct((NN, HID), jnp.float32),
               jax.ShapeDtypeStruct((NN, HID), jnp.float32),
               jax.ShapeDtypeStruct((NN, HID), jnp.int32),
               jax.ShapeDtypeStruct((NN, HID), jnp.float32),
               jax.ShapeDtypeStruct((16, HID), jnp.float32),
               jax.ShapeDtypeStruct((8, HID), jnp.float32)]
    r = lambda v: v.reshape(1, -1)
    return pl.pallas_call(
        _prep_body, grid=(nb,), in_specs=ins, out_specs=outs, out_shape=oshapes,
    )(h, p['emb_h_W'], r(p['emb_h_b']),
      lp['A_W'], r(lp['A_b']), lp['B_W'], r(lp['B_b']),
      lp['D_W'], r(lp['D_b']), lp['E_W'], r(lp['E_b']),
      p['emb_e_W'], r(p['emb_e_b']), lp['C_W'], r(lp['C_b']))


def _edge1_body(e_ref, gs_ref, gd_ref, W1, c1, enew_o, sbsig_o, stats_o, acc):
    i = pl.program_id(0)

    @pl.when(i == 0)
    def _():
        acc[...] = jnp.zeros_like(acc)

    ce = _dot(e_ref[...], W1[...]) + c1[0:1, :]
    dh, bh = _unpack2(gs_ref[...])
    e_new = ce + dh + gd_ref[...]
    enew_o[...] = e_new
    sg = _sigmoid(e_new)
    sbsig_o[:, HID:] = sg
    sbsig_o[:, :HID] = sg * bh
    acc[0:1, :] += jnp.sum(e_new, axis=0, keepdims=True)
    acc[1:2, :] += jnp.sum(e_new * e_new, axis=0, keepdims=True)

    @pl.when(i == pl.num_programs(0) - 1)
    def _():
        stats_o[...] = acc[...]


def _edge1(e, gsrc, gdst, W1, c1):
    ne = e.shape[0]
    nb = ne // EBLK
    blk = lambda s: pl.BlockSpec(s, lambda i: (i, 0))
    full = lambda s: pl.BlockSpec(s, lambda i: (0, 0))
    return pl.pallas_call(
        _edge1_body, grid=(nb,),
        in_specs=[blk((EBLK, 16)), blk((EBLK, HID)), blk((EBLK, HID)),
                  full((16, HID)), full((8, HID))],
        out_specs=[blk((EBLK, HID)), blk((EBLK, 2 * HID)), full((8, HID))],
        out_shape=[jax.ShapeDtypeStruct((ne, HID), jnp.float32),
                   jax.ShapeDtypeStruct((ne, 2 * HID), jnp.float32),
                   jax.ShapeDtypeStruct((8, HID), jnp.float32)],
        scratch_shapes=[pltpu.VMEM((8, HID), jnp.float32)],
    )(e, gsrc, gdst, W1, c1)


def _nodea_body(Ah_ref, numa_ref, dena_ref, numb_ref, denb_ref,
                hpre_o, stats_o, acc):
    i = pl.program_id(0)

    @pl.when(i == 0)
    def _():
        acc[...] = jnp.zeros_like(acc)

    num = numa_ref[...] + numb_ref[...]
    den = dena_ref[...] + denb_ref[...]
    hp = Ah_ref[...] + num / (den + 1e-6)
    hpre_o[...] = hp
    acc[0:1, :] += jnp.sum(hp, axis=0, keepdims=True)
    acc[1:2, :] += jnp.sum(hp * hp, axis=0, keepdims=True)

    @pl.when(i == pl.num_programs(0) - 1)
    def _():
        stats_o[...] = acc[...]


def _nodea(Ah, numa, dena, numb, denb):
    nb = NN // NBLK
    blk = lambda s: pl.BlockSpec(s, lambda i: (i, 0))
    full = lambda s: pl.BlockSpec(s, lambda i: (0, 0))
    return pl.pallas_call(
        _nodea_body, grid=(nb,),
        in_specs=[blk((NBLK, HID))] * 5,
        out_specs=[blk((NBLK, HID)), full((8, HID))],
        out_shape=[jax.ShapeDtypeStruct((NN, HID), jnp.float32),
                   jax.ShapeDtypeStruct((8, HID), jnp.float32)],
        scratch_shapes=[pltpu.VMEM((8, HID), jnp.float32)],
    )(Ah, numa, dena, numb, denb)


def _nodeb_body(hpre_ref, h1_ref, stats_ref, g_ref, b_ref,
                AW, Ab, BW, Bb, DW, Db, EW, Eb,
                h2_o, Ah_o, Tsrc_o, Tdst_o):
    s = stats_ref[0:1, :]
    q = stats_ref[1:2, :]
    mu = s / NN
    var = q / NN - mu * mu
    rstd = jax.lax.rsqrt(var + 1e-5)
    hn = (hpre_ref[...] - mu) * rstd * g_ref[...] + b_ref[...]
    h2 = jnp.maximum(hn, 0.0) + h1_ref[...]
    h2_o[...] = h2
    Ah_o[...] = _dot(h2, AW[...]) + Ab[...]
    dh = _dot(h2, DW[...]) + Db[...]
    bh = _dot(h2, BW[...]) + Bb[...]
    Tsrc_o[...] = _pack2(dh, bh)
    Tdst_o[...] = _dot(h2, EW[...]) + Eb[...]


def _nodeb_mats(hpre, h1, stats, g, b, lp):
    nb = NN // NBLK
    blk = lambda s: pl.BlockSpec(s, lambda i: (i, 0))
    full = lambda s: pl.BlockSpec(s, lambda i: (0, 0))
    w128 = full((HID, HID))
    b128 = full((1, HID))
    r = lambda v: v.reshape(1, -1)
    return pl.pallas_call(
        _nodeb_body, grid=(nb,),
        in_specs=[blk((NBLK, HID)), blk((NBLK, HID)), full((8, HID)),
                  b128, b128] + [w128, b128] * 4,
        out_specs=[blk((NBLK, HID)), blk((NBLK, HID)), blk((NBLK, HID)),
                   blk((NBLK, HID))],
        out_shape=[jax.ShapeDtypeStruct((NN, HID), jnp.float32),
                   jax.ShapeDtypeStruct((NN, HID), jnp.float32),
                   jax.ShapeDtypeStruct((NN, HID), jnp.int32),
                   jax.ShapeDtypeStruct((NN, HID), jnp.float32)],
    )(hpre, h1, stats, r(g), r(b),
      lp['A_W'], r(lp['A_b']), lp['B_W'], r(lp['B_b']),
      lp['D_W'], r(lp['D_b']), lp['E_W'], r(lp['E_b']))


def _nodeb_final_body(hpre_ref, h1_ref, stats_ref, g_ref, b_ref, h2_o):
    s = stats_ref[0:1, :]
    q = stats_ref[1:2, :]
    mu = s / NN
    var = q / NN - mu * mu
    rstd = jax.lax.rsqrt(var + 1e-5)
    hn = (hpre_ref[...] - mu) * rstd * g_ref[...] + b_ref[...]
    h2_o[...] = jnp.maximum(hn, 0.0) + h1_ref[...]


def _nodeb_final(hpre, h1, stats, g, b):
    nb = NN // NBLK
    blk = lambda s: pl.BlockSpec(s, lambda i: (i, 0))
    full = lambda s: pl.BlockSpec(s, lambda i: (0, 0))
    r = lambda v: v.reshape(1, -1)
    return pl.pallas_call(
        _nodeb_final_body, grid=(nb,),
        in_specs=[blk((NBLK, HID)), blk((NBLK, HID)), full((8, HID)),
                  full((1, HID)), full((1, HID))],
        out_specs=blk((NBLK, HID)),
        out_shape=jax.ShapeDtypeStruct((NN, HID), jnp.float32),
    )(hpre, h1, stats, r(g), r(b))


def _edge2_body(enew1_ref, e_ref, gs_ref, gd_ref, statsa_ref, statsb_ref,
                g1_ref, b1_ref, eW, ebias, C2W, C2b, sbsig_o):
    s = statsa_ref[0:1, :] + statsb_ref[0:1, :]
    q = statsa_ref[1:2, :] + statsb_ref[1:2, :]
    mu = s / EE
    var = q / EE - mu * mu
    rstd = jax.lax.rsqrt(var + 1e-5)
    gnorm = (enew1_ref[...] - mu) * rstd * g1_ref[...] + b1_ref[...]
    g1 = jnp.maximum(gnorm, 0.0)
    e1 = _dot(e_ref[...], eW[...]) + ebias[...]
    e_out1 = g1 + e1
    ce2 = _dot(e_out1, C2W[...]) + C2b[...]
    dh, bh = _unpack2(gs_ref[...])
    e_new2 = ce2 + dh + gd_ref[...]
    sg = _sigmoid(e_new2)
    sbsig_o[:, HID:] = sg
    sbsig_o[:, :HID] = sg * bh


def _edge2(enew1, e, gsrc, gdst, statsa, statsb, p, lp1, lp2):
    ne = e.shape[0]
    nb = ne // EBLK
    blk = lambda s: pl.BlockSpec(s, lambda i: (i, 0))
    full = lambda s: pl.BlockSpec(s, lambda i: (0, 0))
    r = lambda v: v.reshape(1, -1)
    return pl.pallas_call(
        _edge2_body, grid=(nb,),
        in_specs=[blk((EBLK, HID)), blk((EBLK, 16)), blk((EBLK, HID)),
                  blk((EBLK, HID)), full((8, HID)), full((8, HID)),
                  full((1, HID)), full((1, HID)), full((16, HID)),
                  full((1, HID)), full((HID, HID)), full((1, HID))],
        out_specs=blk((EBLK, 2 * HID)),
        out_shape=jax.ShapeDtypeStruct((ne, 2 * HID), jnp.float32),
    )(enew1, e, gsrc, gdst, statsa, statsb, r(lp1['bn_e_g']),
      r(lp1['bn_e_b']), p['emb_e_W'], r(p['emb_e_b']),
      lp2['C_W'], r(lp2['C_b']))


_NW = 32
_EPW = EE // _NW
_CHG = 40
_GROWS = _EPW // _CHG
_CHS = 80
_SROWS = (EE // 16) // _CHS


def _sc_mesh():
    return plsc.VectorSubcoreMesh(core_axis_name="c", subcore_axis_name="s")


def _pipeline2(nslots, issue_in, wait_in, issue_out, wait_out):
    def slot(i, b):
        @pl.when(i >= 2)
        def _():
            wait_out(b)

        issue_in(i, b)

        @pl.when(i >= 1)
        def _():
            wait_in(1 - b)
            issue_out(i - 1, 1 - b)

    issue_in(0, 0)

    def body(k, _):
        slot(1 + 2 * k, 1)
        slot(2 + 2 * k, 0)
        return 0

    lax.fori_loop(0, (nslots - 1) // 2, body, 0)
    last = nslots - 1
    if nslots % 2 == 0:
        wait_out(1)
        issue_in(last, 1)
        wait_in(0)
        issue_out(last - 1, 0)
        lastb = 1
    else:
        lastb = 0
    wait_in(lastb)
    issue_out(last, lastb)
    wait_out(1 - lastb)
    wait_out(lastb)


def _gather_sc_body(epw, Tsrc_hbm, Tdst_hbm, src_hbm, dst_hbm,
                    gs_out, gd_out,
                    idxs, idxd, bs0, bs1, bd0, bd1,
                    sgs0, sgs1, sgd0, sgd1, sws0, sws1, swd0, swd1,
                    Ts_sp):
    c = lax.axis_index("c")
    s = lax.axis_index("s")
    wid = s * 2 + c
    base = wid * epw

    @pl.when(s == 0)
    def _():
        pltpu.sync_copy(Tsrc_hbm, Ts_sp)

    pltpu.sync_copy(src_hbm.at[pl.ds(base, epw)], idxs)
    pltpu.sync_copy(dst_hbm.at[pl.ds(base, epw)], idxd)
    plsc.subcore_barrier()
    bs = (bs0, bs1)
    bd = (bd0, bd1)
    sgs = (sgs0, sgs1)
    sgd = (sgd0, sgd1)
    sws = (sws0, sws1)
    swd = (swd0, swd1)

    def issue_in(i, b):
        sl = pl.ds(i * _CHG, _CHG)
        pltpu.async_copy(Ts_sp.at[idxs.at[sl]], bs[b], sgs[b])
        pltpu.async_copy(Tdst_hbm.at[idxd.at[sl]], bd[b], sgd[b])

    def wait_in(b):
        sl = pl.ds(0, _CHG)
        pltpu.make_async_copy(Ts_sp.at[idxs.at[sl]], bs[b], sgs[b]).wait()
        pltpu.make_async_copy(Tdst_hbm.at[idxd.at[sl]], bd[b], sgd[b]).wait()

    def issue_out(i, b):
        off = base + i * _CHG
        pltpu.async_copy(bs[b], gs_out.at[pl.ds(off, _CHG)], sws[b])
        pltpu.async_copy(bd[b], gd_out.at[pl.ds(off, _CHG)], swd[b])

    def wait_out(b):
        pltpu.make_async_copy(bs[b], gs_out.at[pl.ds(0, _CHG)], sws[b]).wait()
        pltpu.make_async_copy(bd[b], gd_out.at[pl.ds(0, _CHG)], swd[b]).wait()

    _pipeline2(epw // _CHG, issue_in, wait_in, issue_out, wait_out)


def _gather(Tsrc, Tdst, src, dst):
    ne = src.shape[0]
    epw = ne // _NW
    f = functools.partial(
        pl.kernel,
        out_type=(jax.ShapeDtypeStruct((ne, HID), jnp.int32),
                  jax.ShapeDtypeStruct((ne, HID), jnp.float32)),
        mesh=_sc_mesh(),
        scratch_types=[pltpu.VMEM((epw,), jnp.int32),
                       pltpu.VMEM((epw,), jnp.int32),
                       pltpu.VMEM((_CHG, HID), jnp.int32),
                       pltpu.VMEM((_CHG, HID), jnp.int32),
                       pltpu.VMEM((_CHG, HID), jnp.float32),
                       pltpu.VMEM((_CHG, HID), jnp.float32)]
                      + [pltpu.SemaphoreType.DMA] * 8
                      + [pltpu.VMEM_SHARED((NN, HID), jnp.int32)],
    )(functools.partial(_gather_sc_body, epw))
    return f(Tsrc, Tdst, src, dst)


_NPAD = 10240


def _scatter_sc_body(eps, valsA_hbm, dstA_hbm,
                     zeros_hbm, out_hbm,
                     ix0, ix1, bv0, bv1,
                     sx0, sx1, sl0, sl1, ss0, ss1, acc):
    c = lax.axis_index("c")
    s = lax.axis_index("s")
    stripe = _NPAD // 16
    pltpu.sync_copy(zeros_hbm.at[pl.ds(s * stripe, stripe)],
                    acc.at[pl.ds(s * stripe, stripe)])
    plsc.subcore_barrier()
    ix = (ix0, ix1)
    bv = (bv0, bv1)
    sx = (sx0, sx1)
    sl = (sl0, sl1)
    ss = (ss0, ss1)
    base = s * eps

    def make_stage(vals_hbm, dst_hbm):
        def issue_in(i, b):
            off = base + i * _CHS
            pltpu.async_copy(dst_hbm.at[pl.ds(off, _CHS)], ix[b], sx[b])
            pltpu.async_copy(
                vals_hbm.at[pl.ds(off, _CHS), pl.ds(c * HID, HID)],
                bv[b], sl[b])

        def wait_in(b):
            pltpu.make_async_copy(
                dst_hbm.at[pl.ds(0, _CHS)], ix[b], sx[b]).wait()
            pltpu.make_async_copy(
                vals_hbm.at[pl.ds(0, _CHS), pl.ds(0, HID)],
                bv[b], sl[b]).wait()

        def issue_out(i, b):
            pltpu.async_copy(bv[b], acc.at[ix[b]], ss[b], add=True)

        def wait_out(b):
            pltpu.make_async_copy(bv[b], acc.at[ix[b]], ss[b]).wait()

        return issue_in, wait_in, issue_out, wait_out

    _pipeline2(eps // _CHS, *make_stage(valsA_hbm, dstA_hbm))
    plsc.subcore_barrier()
    pltpu.sync_copy(acc.at[pl.ds(s * stripe, stripe)],
                    out_hbm.at[c, pl.ds(s * stripe, stripe)])


def _scatter(sbsigA, dstA, zeros):
    eps = dstA.shape[0] // 16
    f = functools.partial(
        pl.kernel,
        out_type=jax.ShapeDtypeStruct((2, _NPAD, HID), jnp.float32),
        mesh=_sc_mesh(),
        scratch_types=[pltpu.VMEM((_CHS,), jnp.int32),
                       pltpu.VMEM((_CHS,), jnp.int32),
                       pltpu.VMEM((_CHS, HID), jnp.float32),
                       pltpu.VMEM((_CHS, HID), jnp.float32)]
                      + [pltpu.SemaphoreType.DMA] * 6
                      + [pltpu.VMEM_SHARED((_NPAD, HID), jnp.float32)],
    )(functools.partial(_scatter_sc_body, eps))
    out = f(sbsigA, dstA, zeros)
    return out[0, :NN], out[1, :NN]


def kernel(h, e, edge_index, params):
    src = edge_index[0]
    dst = edge_index[1]
    lp1, lp2 = params['layers']
    zeros = jnp.zeros((_NPAD, HID), jnp.float32)
    HF = EE // 2
    sA, sB = src[:HF], src[HF:]
    dA, dB = dst[:HF], dst[HF:]
    eA, eB = e[:HF], e[HF:]

    h1, Ah1, Tsrc1, Tdst1, W1, c1 = _prep(h, params, lp1)
    g1a = _gather(Tsrc1, Tdst1, sA, dA)
    g1b = _gather(Tsrc1, Tdst1, sB, dB)
    en1a, sb1a, st1a = _edge1(eA, g1a[0], g1a[1], W1, c1)
    en1b, sb1b, st1b = _edge1(eB, g1b[0], g1b[1], W1, c1)
    n1a, d1a = _scatter(sb1a, dA, zeros)
    n1b, d1b = _scatter(sb1b, dB, zeros)
    hpre1, hstats1 = _nodea(Ah1, n1a, d1a, n1b, d1b)
    h2, Ah2, Tsrc2, Tdst2 = _nodeb_mats(hpre1, h1, hstats1,
                                        lp1['bn_h_g'], lp1['bn_h_b'], lp2)
    g2a = _gather(Tsrc2, Tdst2, sA, dA)
    g2b = _gather(Tsrc2, Tdst2, sB, dB)
    sb2a = _edge2(en1a, eA, g2a[0], g2a[1], st1a, st1b, params, lp1, lp2)
    sb2b = _edge2(en1b, eB, g2b[0], g2b[1], st1a, st1b, params, lp1, lp2)
    n2a, d2a = _scatter(sb2a, dA, zeros)
    n2b, d2b = _scatter(sb2b, dB, zeros)
    hpre2, hstats2 = _nodea(Ah2, n2a, d2a, n2b, d2b)
    return _nodeb_final(hpre2, h2, hstats2, lp2['bn_h_g'], lp2['bn_h_b'])
